# baseline (speedup 1.0000x reference)
import jax
import jax.numpy as jnp
from jax import lax
from jax.experimental import pallas as pl
from jax.experimental.pallas import tpu as pltpu
from jax.experimental.pallas import tpu_sc as plsc

_N = 10000
_E = 320000
_D = 128
_G = 64

_NS = 16
_NC = 2
_LANES = 16

_N_PAD = 10240
_NPT = _N_PAD // _NS

_E_PT16 = _E // _NS
_E_PT32 = _E // (_NS * _NC)
_CH = 200
_NCHUNK = _E_PT32 // _CH

_PNPT = _N_PAD // (_NS * _NC)


def _mlp_body(h_ref, wa_ref, ba_ref, wb_ref, bb_ref, o_ref):
    h = jnp.maximum(jnp.dot(h_ref[...], wa_ref[...]) + ba_ref[...], 0.0)
    h = jnp.dot(h, wb_ref[...]) + bb_ref[...]
    o_ref[...] = jnp.maximum(h, 0.0)


def _mlp1_body(s_ref, wa_ref, ba_ref, wb_ref, bb_ref, o_ref):
    h = jnp.broadcast_to(s_ref[...], (s_ref.shape[0], _D))
    h = jnp.maximum(jnp.dot(h, wa_ref[...]) + ba_ref[...], 0.0)
    h = jnp.dot(h, wb_ref[...]) + bb_ref[...]
    o_ref[...] = jnp.maximum(h, 0.0)


def _mlp_merge_body(p0_ref, p1_ref, x_ref, wa_ref, ba_ref, wb_ref, bb_ref,
                    o_ref):
    h = (p0_ref[...] + p1_ref[...]) - x_ref[...]
    h = jnp.maximum(jnp.dot(h, wa_ref[...]) + ba_ref[...], 0.0)
    h = jnp.dot(h, wb_ref[...]) + bb_ref[...]
    o_ref[...] = jnp.maximum(h, 0.0)


_GRID = 10
_BLK = _N_PAD // _GRID

_W_SPECS = [
    pl.BlockSpec((_D, _D), lambda i: (0, 0)),
    pl.BlockSpec((1, _D), lambda i: (0, 0)),
    pl.BlockSpec((_D, _D), lambda i: (0, 0)),
    pl.BlockSpec((1, _D), lambda i: (0, 0)),
]


def _mlp1(s2d, Wa, ba, Wb, bb):
    return pl.pallas_call(
        _mlp1_body,
        grid=(_GRID,),
        in_specs=[pl.BlockSpec((_BLK, 1), lambda i: (i, 0))] + _W_SPECS,
        out_specs=pl.BlockSpec((_BLK, _D), lambda i: (i, 0)),
        out_shape=jax.ShapeDtypeStruct((_N_PAD, _D), jnp.float32),
    )(s2d, Wa, ba.reshape(1, _D), Wb, bb.reshape(1, _D))


def _mlp_merge(hp, x, Wa, ba, Wb, bb):
    return pl.pallas_call(
        _mlp_merge_body,
        grid=(_GRID,),
        in_specs=[
            pl.BlockSpec((_BLK, _D), lambda i: (i, 0)),
            pl.BlockSpec((_BLK, _D), lambda i: (i + _GRID, 0)),
            pl.BlockSpec((_BLK, _D), lambda i: (i, 0)),
        ] + _W_SPECS,
        out_specs=pl.BlockSpec((_BLK, _D), lambda i: (i, 0)),
        out_shape=jax.ShapeDtypeStruct((_N_PAD, _D), jnp.float32),
    )(hp, hp, x, Wa, ba.reshape(1, _D), Wb, bb.reshape(1, _D))


def _deg_body(ei_h, s_h, sh_s, dst_v, vals_v, node_v, sem1):
    cid = lax.axis_index("c")
    wid = lax.axis_index("s")
    nbase = wid * _NPT

    d1 = pltpu.async_copy(ei_h.at[pl.ds(_E + wid * _E_PT16, _E_PT16)],
                          dst_v, sem1)
    one16 = jnp.full((_LANES,), 1.0, jnp.float32)
    for i in range(_NPT // _LANES):
        node_v[pl.ds(i * _LANES, _LANES)] = one16

    def fill(j, carry):
        vals_v[pl.ds(j * _LANES, _LANES)] = one16
        return carry
    lax.fori_loop(0, _E_PT16 // _LANES, fill, 0)

    pltpu.sync_copy(node_v, sh_s.at[pl.ds(nbase, _NPT)])
    d1.wait()
    plsc.subcore_barrier()
    pltpu.sync_copy(vals_v, sh_s.at[dst_v], add=True)
    plsc.subcore_barrier()

    @pl.when(cid == 0)
    def _():
        pltpu.sync_copy(sh_s.at[pl.ds(nbase, _NPT)],
                        s_h.at[pl.ds(nbase, _NPT)])


_deg_call = pl.kernel(
    _deg_body,
    out_type=jax.ShapeDtypeStruct((_N_PAD,), jnp.float32),
    mesh=plsc.VectorSubcoreMesh(core_axis_name="c", subcore_axis_name="s"),
    scratch_types=[
        pltpu.VMEM_SHARED((_N_PAD,), jnp.float32),
        pltpu.VMEM((_E_PT16,), jnp.int32),
        pltpu.VMEM((_E_PT16,), jnp.float32),
        pltpu.VMEM((_NPT,), jnp.float32),
        pltpu.SemaphoreType.DMA,
    ],
)


def _aggr_body(x_h, ei_h, hp_h, slab, srcb, dstb, rows_b, sem1):
    cid = lax.axis_index("c")
    wid = lax.axis_index("s")
    tid = cid * _NS + wid
    ebase = tid * _E_PT32
    nbase = wid * _NPT

    pltpu.sync_copy(x_h.at[pl.ds(nbase, _NPT)], slab.at[pl.ds(nbase, _NPT)])
    plsc.subcore_barrier()

    def chunk(k, carry):
        off = ebase + k * _CH
        pltpu.sync_copy(ei_h.at[pl.ds(off, _CH)], srcb)
        pltpu.sync_copy(ei_h.at[pl.ds(_E + off, _CH)], dstb)
        pltpu.async_copy(x_h.at[srcb], rows_b, sem1).wait()
        pltpu.sync_copy(rows_b, slab.at[dstb], add=True)
        return carry
    lax.fori_loop(0, _NCHUNK, chunk, 0)
    plsc.subcore_barrier()

    pltpu.sync_copy(slab.at[pl.ds(nbase, _NPT)],
                    hp_h.at[pl.ds(cid * _N_PAD + nbase, _NPT)])


_aggr_call = pl.kernel(
    _aggr_body,
    out_type=jax.ShapeDtypeStruct((_NC * _N_PAD, _D), jnp.float32),
    mesh=plsc.VectorSubcoreMesh(core_axis_name="c", subcore_axis_name="s"),
    scratch_types=[
        pltpu.VMEM_SHARED((_N_PAD, _D), jnp.float32),
        pltpu.VMEM((_CH,), jnp.int32),
        pltpu.VMEM((_CH,), jnp.int32),
        pltpu.VMEM((_CH, _D), jnp.float32),
        pltpu.SemaphoreType.DMA,
    ],
)


def _pool_body(x_h, bat_h, ps_h, cnt_h, pslab, csh, xrows, batb, onesb, zb):
    cid = lax.axis_index("c")
    wid = lax.axis_index("s")
    tid = cid * _NS + wid
    rbase = tid * _PNPT

    z16 = jnp.zeros((_LANES,), jnp.float32)
    for r in range(_LANES):
        for i in range(_D // _LANES):
            zb[r, pl.ds(i * _LANES, _LANES)] = z16
    one16 = jnp.full((_LANES,), 1.0, jnp.float32)
    for i in range(_PNPT // _LANES):
        onesb[pl.ds(i * _LANES, _LANES)] = one16

    pltpu.sync_copy(x_h.at[pl.ds(rbase, _PNPT)], xrows)
    pltpu.sync_copy(bat_h.at[pl.ds(rbase, _PNPT)], batb)
    @pl.when(wid < 8)
    def _():
        pltpu.sync_copy(zb, pslab.at[pl.ds(wid * _LANES, _LANES)])

    @pl.when(wid == 8)
    def _():
        pltpu.sync_copy(zb.at[0], csh.at[pl.ds(0, _D)])
    plsc.subcore_barrier()

    pltpu.sync_copy(xrows, pslab.at[batb], add=True)
    plsc.subcore_barrier()
    pltpu.sync_copy(onesb, csh.at[batb], add=True)
    plsc.subcore_barrier()

    @pl.when(wid < 8)
    def _():
        pltpu.sync_copy(pslab.at[pl.ds(wid * _LANES, _LANES)],
                        ps_h.at[pl.ds(cid * _D + wid * _LANES, _LANES)])

    @pl.when(wid == 8)
    def _():
        pltpu.sync_copy(csh, cnt_h.at[pl.ds(cid * _D, _D)])


_pool_call = pl.kernel(
    _pool_body,
    out_type=[jax.ShapeDtypeStruct((_NC * _D, _D), jnp.float32),
              jax.ShapeDtypeStruct((_NC * _D,), jnp.float32)],
    mesh=plsc.VectorSubcoreMesh(core_axis_name="c", subcore_axis_name="s"),
    scratch_types=[
        pltpu.VMEM_SHARED((_D, _D), jnp.float32),
        pltpu.VMEM_SHARED((_D,), jnp.float32),
        pltpu.VMEM((_PNPT, _D), jnp.float32),
        pltpu.VMEM((_PNPT,), jnp.int32),
        pltpu.VMEM((_PNPT,), jnp.float32),
        pltpu.VMEM((_LANES, _D), jnp.float32),
    ],
)


@jax.jit
def kernel(W1a, b1a, W1b, b1b, W2a, b2a, W2b, b2b, W3a, b3a, W3b, b3b,
           Wl, bl, edge_index, batch):
    ei = edge_index.reshape(2 * _E)
    bat = jnp.pad(batch, (0, _N_PAD - _N), constant_values=_G)

    s = _deg_call(ei)
    x1 = _mlp1(s.reshape(_N_PAD, 1), W1a, b1a, W1b, b1b)
    hp = _aggr_call(x1, ei)
    x2 = _mlp_merge(hp, x1, W2a, b2a, W2b, b2b)
    hp = _aggr_call(x2, ei)
    x3 = _mlp_merge(hp, x2, W3a, b3a, W3b, b3b)
    ps, cnt = _pool_call(x3, bat)

    sums = ps[0:_G] + ps[_D:_D + _G]
    counts = (cnt[0:_G] + cnt[_D:_D + _G])[:, None]
    pooled = sums / jnp.maximum(counts, 1.0)
    return jnp.dot(pooled, Wl) + bl


# speedup vs baseline: 1.1411x; 1.1411x over previous
import jax
import jax.numpy as jnp
from jax import lax
from jax.experimental import pallas as pl
from jax.experimental.pallas import tpu as pltpu
from jax.experimental.pallas import tpu_sc as plsc

_N = 10000
_E = 320000
_D = 128
_G = 64

_NS = 16
_NC = 2
_LANES = 16

_N_PAD = 10240
_NPT = _N_PAD // _NS

_E_PT16 = _E // _NS
_E_PT32 = _E // (_NS * _NC)
_CH = 80
_NCHUNK = _E_PT32 // _CH

_PNPT = _N_PAD // (_NS * _NC)


def _mlp_body(h_ref, wa_ref, ba_ref, wb_ref, bb_ref, o_ref):
    h = jnp.maximum(jnp.dot(h_ref[...], wa_ref[...]) + ba_ref[...], 0.0)
    h = jnp.dot(h, wb_ref[...]) + bb_ref[...]
    o_ref[...] = jnp.maximum(h, 0.0)


def _mlp1_body(s_ref, wa_ref, ba_ref, wb_ref, bb_ref, o_ref):
    h = jnp.broadcast_to(s_ref[...], (s_ref.shape[0], _D))
    h = jnp.maximum(jnp.dot(h, wa_ref[...]) + ba_ref[...], 0.0)
    h = jnp.dot(h, wb_ref[...]) + bb_ref[...]
    o_ref[...] = jnp.maximum(h, 0.0)


def _mlp_merge_body(p0_ref, p1_ref, x_ref, wa_ref, ba_ref, wb_ref, bb_ref,
                    o_ref):
    h = (p0_ref[...] + p1_ref[...]) - x_ref[...]
    h = jnp.maximum(jnp.dot(h, wa_ref[...]) + ba_ref[...], 0.0)
    h = jnp.dot(h, wb_ref[...]) + bb_ref[...]
    o_ref[...] = jnp.maximum(h, 0.0)


_GRID = 10
_BLK = _N_PAD // _GRID

_W_SPECS = [
    pl.BlockSpec((_D, _D), lambda i: (0, 0)),
    pl.BlockSpec((1, _D), lambda i: (0, 0)),
    pl.BlockSpec((_D, _D), lambda i: (0, 0)),
    pl.BlockSpec((1, _D), lambda i: (0, 0)),
]


def _mlp1(s2d, Wa, ba, Wb, bb):
    return pl.pallas_call(
        _mlp1_body,
        grid=(_GRID,),
        in_specs=[pl.BlockSpec((_BLK, 1), lambda i: (i, 0))] + _W_SPECS,
        out_specs=pl.BlockSpec((_BLK, _D), lambda i: (i, 0)),
        out_shape=jax.ShapeDtypeStruct((_N_PAD, _D), jnp.float32),
    )(s2d, Wa, ba.reshape(1, _D), Wb, bb.reshape(1, _D))


def _mlp_merge(hp, x, Wa, ba, Wb, bb):
    return pl.pallas_call(
        _mlp_merge_body,
        grid=(_GRID,),
        in_specs=[
            pl.BlockSpec((_BLK, _D), lambda i: (i, 0)),
            pl.BlockSpec((_BLK, _D), lambda i: (i + _GRID, 0)),
            pl.BlockSpec((_BLK, _D), lambda i: (i, 0)),
        ] + _W_SPECS,
        out_specs=pl.BlockSpec((_BLK, _D), lambda i: (i, 0)),
        out_shape=jax.ShapeDtypeStruct((_N_PAD, _D), jnp.float32),
    )(hp, hp, x, Wa, ba.reshape(1, _D), Wb, bb.reshape(1, _D))


def _deg_body(ei_h, s_h, sh_s, dst_v, vals_v, node_v, sem1):
    cid = lax.axis_index("c")
    wid = lax.axis_index("s")
    nbase = wid * _NPT

    d1 = pltpu.async_copy(ei_h.at[pl.ds(_E + wid * _E_PT16, _E_PT16)],
                          dst_v, sem1)
    one16 = jnp.full((_LANES,), 1.0, jnp.float32)
    for i in range(_NPT // _LANES):
        node_v[pl.ds(i * _LANES, _LANES)] = one16

    def fill(j, carry):
        vals_v[pl.ds(j * _LANES, _LANES)] = one16
        return carry
    lax.fori_loop(0, _E_PT16 // _LANES, fill, 0)

    pltpu.sync_copy(node_v, sh_s.at[pl.ds(nbase, _NPT)])
    d1.wait()
    plsc.subcore_barrier()
    pltpu.sync_copy(vals_v, sh_s.at[dst_v], add=True)
    plsc.subcore_barrier()

    @pl.when(cid == 0)
    def _():
        pltpu.sync_copy(sh_s.at[pl.ds(nbase, _NPT)],
                        s_h.at[pl.ds(nbase, _NPT)])


_deg_call = pl.kernel(
    _deg_body,
    out_type=jax.ShapeDtypeStruct((_N_PAD,), jnp.float32),
    mesh=plsc.VectorSubcoreMesh(core_axis_name="c", subcore_axis_name="s"),
    scratch_types=[
        pltpu.VMEM_SHARED((_N_PAD,), jnp.float32),
        pltpu.VMEM((_E_PT16,), jnp.int32),
        pltpu.VMEM((_E_PT16,), jnp.float32),
        pltpu.VMEM((_NPT,), jnp.float32),
        pltpu.SemaphoreType.DMA,
    ],
)


def _aggr_body(x_h, ei_h, hp_h, slab, srcall, dstb0, dstb1, rows0, rows1,
               gsem, ssem, lsem):
    cid = lax.axis_index("c")
    wid = lax.axis_index("s")
    tid = cid * _NS + wid
    ebase = tid * _E_PT32
    nbase = wid * _NPT

    dsrc = pltpu.async_copy(ei_h.at[pl.ds(ebase, _E_PT32)], srcall, lsem)
    pltpu.sync_copy(x_h.at[pl.ds(nbase, _NPT)], slab.at[pl.ds(nbase, _NPT)])
    dsrc.wait()
    plsc.subcore_barrier()

    def pair(j, carry):
        off0 = ebase + (2 * j) * _CH
        off1 = off0 + _CH
        s0 = (2 * j) * _CH
        pltpu.sync_copy(ei_h.at[pl.ds(_E + off0, _CH)], dstb0)
        g0 = pltpu.async_copy(
            x_h.at[srcall.at[pl.ds(s0, _CH)]], rows0, gsem)
        pltpu.sync_copy(ei_h.at[pl.ds(_E + off1, _CH)], dstb1)
        g1 = pltpu.async_copy(
            x_h.at[srcall.at[pl.ds(s0 + _CH, _CH)]], rows1, gsem)
        g0.wait()
        w0 = pltpu.async_copy(rows0, slab.at[dstb0], ssem, add=True)
        g1.wait()
        w1 = pltpu.async_copy(rows1, slab.at[dstb1], ssem, add=True)
        w0.wait()
        w1.wait()
        return carry
    lax.fori_loop(0, _NCHUNK // 2, pair, 0)

    offt = ebase + (_NCHUNK - 1) * _CH
    pltpu.sync_copy(ei_h.at[pl.ds(_E + offt, _CH)], dstb0)
    pltpu.async_copy(
        x_h.at[srcall.at[pl.ds((_NCHUNK - 1) * _CH, _CH)]], rows0,
        gsem).wait()
    pltpu.sync_copy(rows0, slab.at[dstb0], add=True)
    plsc.subcore_barrier()

    pltpu.sync_copy(slab.at[pl.ds(nbase, _NPT)],
                    hp_h.at[pl.ds(cid * _N_PAD + nbase, _NPT)])


_aggr_call = pl.kernel(
    _aggr_body,
    out_type=jax.ShapeDtypeStruct((_NC * _N_PAD, _D), jnp.float32),
    mesh=plsc.VectorSubcoreMesh(core_axis_name="c", subcore_axis_name="s"),
    scratch_types=[
        pltpu.VMEM_SHARED((_N_PAD, _D), jnp.float32),
        pltpu.VMEM((_E_PT32,), jnp.int32),
        pltpu.VMEM((_CH,), jnp.int32),
        pltpu.VMEM((_CH,), jnp.int32),
        pltpu.VMEM((_CH, _D), jnp.float32),
        pltpu.VMEM((_CH, _D), jnp.float32),
        pltpu.SemaphoreType.DMA,
        pltpu.SemaphoreType.DMA,
        pltpu.SemaphoreType.DMA,
    ],
)


def _pool_body(x_h, bat_h, ps_h, cnt_h, pslab, csh, xrows, batb, onesb, zb):
    cid = lax.axis_index("c")
    wid = lax.axis_index("s")
    tid = cid * _NS + wid
    rbase = tid * _PNPT

    z16 = jnp.zeros((_LANES,), jnp.float32)
    for r in range(_LANES):
        for i in range(_D // _LANES):
            zb[r, pl.ds(i * _LANES, _LANES)] = z16
    one16 = jnp.full((_LANES,), 1.0, jnp.float32)
    for i in range(_PNPT // _LANES):
        onesb[pl.ds(i * _LANES, _LANES)] = one16

    pltpu.sync_copy(x_h.at[pl.ds(rbase, _PNPT)], xrows)
    pltpu.sync_copy(bat_h.at[pl.ds(rbase, _PNPT)], batb)
    @pl.when(wid < 8)
    def _():
        pltpu.sync_copy(zb, pslab.at[pl.ds(wid * _LANES, _LANES)])

    @pl.when(wid == 8)
    def _():
        pltpu.sync_copy(zb.at[0], csh.at[pl.ds(0, _D)])
    plsc.subcore_barrier()

    pltpu.sync_copy(xrows, pslab.at[batb], add=True)
    plsc.subcore_barrier()
    pltpu.sync_copy(onesb, csh.at[batb], add=True)
    plsc.subcore_barrier()

    @pl.when(wid < 8)
    def _():
        pltpu.sync_copy(pslab.at[pl.ds(wid * _LANES, _LANES)],
                        ps_h.at[pl.ds(cid * _D + wid * _LANES, _LANES)])

    @pl.when(wid == 8)
    def _():
        pltpu.sync_copy(csh, cnt_h.at[pl.ds(cid * _D, _D)])


_pool_call = pl.kernel(
    _pool_body,
    out_type=[jax.ShapeDtypeStruct((_NC * _D, _D), jnp.float32),
              jax.ShapeDtypeStruct((_NC * _D,), jnp.float32)],
    mesh=plsc.VectorSubcoreMesh(core_axis_name="c", subcore_axis_name="s"),
    scratch_types=[
        pltpu.VMEM_SHARED((_D, _D), jnp.float32),
        pltpu.VMEM_SHARED((_D,), jnp.float32),
        pltpu.VMEM((_PNPT, _D), jnp.float32),
        pltpu.VMEM((_PNPT,), jnp.int32),
        pltpu.VMEM((_PNPT,), jnp.float32),
        pltpu.VMEM((_LANES, _D), jnp.float32),
    ],
)


@jax.jit
def kernel(W1a, b1a, W1b, b1b, W2a, b2a, W2b, b2b, W3a, b3a, W3b, b3b,
           Wl, bl, edge_index, batch):
    ei = edge_index.reshape(2 * _E)
    bat = jnp.pad(batch, (0, _N_PAD - _N), constant_values=_G)

    s = _deg_call(ei)
    x1 = _mlp1(s.reshape(_N_PAD, 1), W1a, b1a, W1b, b1b)
    hp = _aggr_call(x1, ei)
    x2 = _mlp_merge(hp, x1, W2a, b2a, W2b, b2b)
    hp = _aggr_call(x2, ei)
    x3 = _mlp_merge(hp, x2, W3a, b3a, W3b, b3b)
    ps, cnt = _pool_call(x3, bat)

    sums = ps[0:_G] + ps[_D:_D + _G]
    counts = (cnt[0:_G] + cnt[_D:_D + _G])[:, None]
    pooled = sums / jnp.maximum(counts, 1.0)
    return jnp.dot(pooled, Wl) + bl


# speedup vs baseline: 1.1421x; 1.0009x over previous
import jax
import jax.numpy as jnp
from jax import lax
from jax.experimental import pallas as pl
from jax.experimental.pallas import tpu as pltpu
from jax.experimental.pallas import tpu_sc as plsc

_N = 10000
_E = 320000
_D = 128
_G = 64

_NS = 16
_NC = 2
_LANES = 16

_N_PAD = 10240
_NPT = _N_PAD // _NS

_E_PT16 = _E // _NS
_E_PT32 = _E // (_NS * _NC)
_CH = 80
_NCHUNK = _E_PT32 // _CH

_PNPT = _N_PAD // (_NS * _NC)


def _mlp_body(h_ref, wa_ref, ba_ref, wb_ref, bb_ref, o_ref):
    h = jnp.maximum(jnp.dot(h_ref[...], wa_ref[...]) + ba_ref[...], 0.0)
    h = jnp.dot(h, wb_ref[...]) + bb_ref[...]
    o_ref[...] = jnp.maximum(h, 0.0)


def _mlp1_body(s_ref, wa_ref, ba_ref, wb_ref, bb_ref, o_ref):
    h = jnp.broadcast_to(s_ref[...], (s_ref.shape[0], _D))
    h = jnp.maximum(jnp.dot(h, wa_ref[...]) + ba_ref[...], 0.0)
    h = jnp.dot(h, wb_ref[...]) + bb_ref[...]
    o_ref[...] = jnp.maximum(h, 0.0)


def _mlp_merge_body(p0_ref, p1_ref, x_ref, wa_ref, ba_ref, wb_ref, bb_ref,
                    o_ref):
    h = (p0_ref[...] + p1_ref[...]) - x_ref[...]
    h = jnp.maximum(jnp.dot(h, wa_ref[...]) + ba_ref[...], 0.0)
    h = jnp.dot(h, wb_ref[...]) + bb_ref[...]
    o_ref[...] = jnp.maximum(h, 0.0)


_GRID = 10
_BLK = _N_PAD // _GRID

_W_SPECS = [
    pl.BlockSpec((_D, _D), lambda i: (0, 0)),
    pl.BlockSpec((1, _D), lambda i: (0, 0)),
    pl.BlockSpec((_D, _D), lambda i: (0, 0)),
    pl.BlockSpec((1, _D), lambda i: (0, 0)),
]


def _mlp1(s2d, Wa, ba, Wb, bb):
    return pl.pallas_call(
        _mlp1_body,
        grid=(_GRID,),
        in_specs=[pl.BlockSpec((_BLK, 1), lambda i: (i, 0))] + _W_SPECS,
        out_specs=pl.BlockSpec((_BLK, _D), lambda i: (i, 0)),
        out_shape=jax.ShapeDtypeStruct((_N_PAD, _D), jnp.float32),
    )(s2d, Wa, ba.reshape(1, _D), Wb, bb.reshape(1, _D))


def _mlp_merge(hp, x, Wa, ba, Wb, bb):
    return pl.pallas_call(
        _mlp_merge_body,
        grid=(_GRID,),
        in_specs=[
            pl.BlockSpec((_BLK, _D), lambda i: (i, 0)),
            pl.BlockSpec((_BLK, _D), lambda i: (i + _GRID, 0)),
            pl.BlockSpec((_BLK, _D), lambda i: (i, 0)),
        ] + _W_SPECS,
        out_specs=pl.BlockSpec((_BLK, _D), lambda i: (i, 0)),
        out_shape=jax.ShapeDtypeStruct((_N_PAD, _D), jnp.float32),
    )(hp, hp, x, Wa, ba.reshape(1, _D), Wb, bb.reshape(1, _D))


def _deg_body(ei_h, s_h, sh_s, dst_v, vals_v, node_v, sem1):
    cid = lax.axis_index("c")
    wid = lax.axis_index("s")
    nbase = wid * _NPT

    d1 = pltpu.async_copy(ei_h.at[pl.ds(_E + wid * _E_PT16, _E_PT16)],
                          dst_v, sem1)
    one16 = jnp.full((_LANES,), 1.0, jnp.float32)
    for i in range(_NPT // _LANES):
        node_v[pl.ds(i * _LANES, _LANES)] = one16

    def fill(j, carry):
        vals_v[pl.ds(j * _LANES, _LANES)] = one16
        return carry
    lax.fori_loop(0, _E_PT16 // _LANES, fill, 0)

    pltpu.sync_copy(node_v, sh_s.at[pl.ds(nbase, _NPT)])
    d1.wait()
    plsc.subcore_barrier()
    pltpu.sync_copy(vals_v, sh_s.at[dst_v], add=True)
    plsc.subcore_barrier()

    @pl.when(cid == 0)
    def _():
        pltpu.sync_copy(sh_s.at[pl.ds(nbase, _NPT)],
                        s_h.at[pl.ds(nbase, _NPT)])


_deg_call = pl.kernel(
    _deg_body,
    out_type=jax.ShapeDtypeStruct((_N_PAD,), jnp.float32),
    mesh=plsc.VectorSubcoreMesh(core_axis_name="c", subcore_axis_name="s", num_cores=_NC, num_subcores=_NS),
    scratch_types=[
        pltpu.VMEM_SHARED((_N_PAD,), jnp.float32),
        pltpu.VMEM((_E_PT16,), jnp.int32),
        pltpu.VMEM((_E_PT16,), jnp.float32),
        pltpu.VMEM((_NPT,), jnp.float32),
        pltpu.SemaphoreType.DMA,
    ],
)


def _aggr_body(x_h, ei_h, hp_h, slab, srcall, dstb0, dstb1, rows0, rows1,
               gsem, ssem, lsem):
    cid = lax.axis_index("c")
    wid = lax.axis_index("s")
    tid = cid * _NS + wid
    ebase = tid * _E_PT32
    nbase = wid * _NPT

    dsrc = pltpu.async_copy(ei_h.at[pl.ds(ebase, _E_PT32)], srcall, lsem)
    pltpu.sync_copy(x_h.at[pl.ds(nbase, _NPT)], slab.at[pl.ds(nbase, _NPT)])
    dsrc.wait()
    plsc.subcore_barrier()

    def pair(j, carry):
        off0 = ebase + (2 * j) * _CH
        off1 = off0 + _CH
        s0 = (2 * j) * _CH
        pltpu.sync_copy(ei_h.at[pl.ds(_E + off0, _CH)], dstb0)
        g0 = pltpu.async_copy(
            x_h.at[srcall.at[pl.ds(s0, _CH)]], rows0, gsem)
        pltpu.sync_copy(ei_h.at[pl.ds(_E + off1, _CH)], dstb1)
        g1 = pltpu.async_copy(
            x_h.at[srcall.at[pl.ds(s0 + _CH, _CH)]], rows1, gsem)
        g0.wait()
        w0 = pltpu.async_copy(rows0, slab.at[dstb0], ssem, add=True)
        g1.wait()
        w1 = pltpu.async_copy(rows1, slab.at[dstb1], ssem, add=True)
        w0.wait()
        w1.wait()
        return carry
    lax.fori_loop(0, _NCHUNK // 2, pair, 0)

    offt = ebase + (_NCHUNK - 1) * _CH
    pltpu.sync_copy(ei_h.at[pl.ds(_E + offt, _CH)], dstb0)
    pltpu.async_copy(
        x_h.at[srcall.at[pl.ds((_NCHUNK - 1) * _CH, _CH)]], rows0,
        gsem).wait()
    pltpu.sync_copy(rows0, slab.at[dstb0], add=True)
    plsc.subcore_barrier()

    pltpu.sync_copy(slab.at[pl.ds(nbase, _NPT)],
                    hp_h.at[pl.ds(cid * _N_PAD + nbase, _NPT)])


_aggr_call = pl.kernel(
    _aggr_body,
    out_type=jax.ShapeDtypeStruct((_NC * _N_PAD, _D), jnp.float32),
    mesh=plsc.VectorSubcoreMesh(core_axis_name="c", subcore_axis_name="s", num_cores=_NC, num_subcores=_NS),
    scratch_types=[
        pltpu.VMEM_SHARED((_N_PAD, _D), jnp.float32),
        pltpu.VMEM((_E_PT32,), jnp.int32),
        pltpu.VMEM((_CH,), jnp.int32),
        pltpu.VMEM((_CH,), jnp.int32),
        pltpu.VMEM((_CH, _D), jnp.float32),
        pltpu.VMEM((_CH, _D), jnp.float32),
        pltpu.SemaphoreType.DMA,
        pltpu.SemaphoreType.DMA,
        pltpu.SemaphoreType.DMA,
    ],
)


def _pool_body(x_h, bat_h, ps_h, cnt_h, pslab, csh, xrows, batb, onesb, zb):
    cid = lax.axis_index("c")
    wid = lax.axis_index("s")
    tid = cid * _NS + wid
    rbase = tid * _PNPT

    z16 = jnp.zeros((_LANES,), jnp.float32)
    for r in range(_LANES):
        for i in range(_D // _LANES):
            zb[r, pl.ds(i * _LANES, _LANES)] = z16
    one16 = jnp.full((_LANES,), 1.0, jnp.float32)
    for i in range(_PNPT // _LANES):
        onesb[pl.ds(i * _LANES, _LANES)] = one16

    pltpu.sync_copy(x_h.at[pl.ds(rbase, _PNPT)], xrows)
    pltpu.sync_copy(bat_h.at[pl.ds(rbase, _PNPT)], batb)
    @pl.when(wid < 8)
    def _():
        pltpu.sync_copy(zb, pslab.at[pl.ds(wid * _LANES, _LANES)])

    @pl.when(wid == 8)
    def _():
        pltpu.sync_copy(zb.at[0], csh.at[pl.ds(0, _D)])
    plsc.subcore_barrier()

    pltpu.sync_copy(xrows, pslab.at[batb], add=True)
    plsc.subcore_barrier()
    pltpu.sync_copy(onesb, csh.at[batb], add=True)
    plsc.subcore_barrier()

    @pl.when(wid < 8)
    def _():
        pltpu.sync_copy(pslab.at[pl.ds(wid * _LANES, _LANES)],
                        ps_h.at[pl.ds(cid * _D + wid * _LANES, _LANES)])

    @pl.when(wid == 8)
    def _():
        pltpu.sync_copy(csh, cnt_h.at[pl.ds(cid * _D, _D)])


_pool_call = pl.kernel(
    _pool_body,
    out_type=[jax.ShapeDtypeStruct((_NC * _D, _D), jnp.float32),
              jax.ShapeDtypeStruct((_NC * _D,), jnp.float32)],
    mesh=plsc.VectorSubcoreMesh(core_axis_name="c", subcore_axis_name="s", num_cores=_NC, num_subcores=_NS),
    scratch_types=[
        pltpu.VMEM_SHARED((_D, _D), jnp.float32),
        pltpu.VMEM_SHARED((_D,), jnp.float32),
        pltpu.VMEM((_PNPT, _D), jnp.float32),
        pltpu.VMEM((_PNPT,), jnp.int32),
        pltpu.VMEM((_PNPT,), jnp.float32),
        pltpu.VMEM((_LANES, _D), jnp.float32),
    ],
)


@jax.jit
def kernel(W1a, b1a, W1b, b1b, W2a, b2a, W2b, b2b, W3a, b3a, W3b, b3b,
           Wl, bl, edge_index, batch):
    ei = edge_index.reshape(2 * _E)
    bat = jnp.pad(batch, (0, _N_PAD - _N), constant_values=_G)

    s = _deg_call(ei)
    x1 = _mlp1(s.reshape(_N_PAD, 1), W1a, b1a, W1b, b1b)
    hp = _aggr_call(x1, ei)
    x2 = _mlp_merge(hp, x1, W2a, b2a, W2b, b2b)
    hp = _aggr_call(x2, ei)
    x3 = _mlp_merge(hp, x2, W3a, b3a, W3b, b3b)
    ps, cnt = _pool_call(x3, bat)

    sums = ps[0:_G] + ps[_D:_D + _G]
    counts = (cnt[0:_G] + cnt[_D:_D + _G])[:, None]
    pooled = sums / jnp.maximum(counts, 1.0)
    return jnp.dot(pooled, Wl) + bl
